# initial kernel scaffold (unmeasured)
import jax
import jax.numpy as jnp
from jax import lax
from jax.experimental import pallas as pl
from jax.experimental.pallas import tpu as pltpu


def kernel(Q, K, V):
    b, s, h, d = Q.shape
    scale = d ** -0.5

    def body(q_ref, k_ref, v_ref, out_ref, k_other, v_other, send_sems, recv_sems):
        my_x = lax.axis_index("x")
        my_y = lax.axis_index("y")
        peer = (1 - my_x, my_y)

        barrier_sem = pltpu.get_barrier_semaphore()
        pl.semaphore_signal(
            barrier_sem, inc=1, device_id=peer,
            device_id_type=pl.DeviceIdType.MESH,
        )
        pl.semaphore_wait(barrier_sem, 1)

        rdma_k = pltpu.make_async_remote_copy(
            src_ref=k_ref, dst_ref=k_other,
            send_sem=send_sems.at[0], recv_sem=recv_sems.at[0],
            device_id=peer, device_id_type=pl.DeviceIdType.MESH,
        )
        rdma_v = pltpu.make_async_remote_copy(
            src_ref=v_ref, dst_ref=v_other,
            send_sem=send_sems.at[1], recv_sem=recv_sems.at[1],
            device_id=peer, device_id_type=pl.DeviceIdType.MESH,
        )
        rdma_k.start()
        rdma_v.start()
        rdma_k.wait()
        rdma_v.wait()

        dn_qk = (((1,), (1,)), ((), ()))
        dn_pv = (((1,), (0,)), ((), ()))
        for head in range(h):
            q = q_ref[0, :, head, :]
            s1 = lax.dot_general(q, k_ref[0, :, head, :], dn_qk,
                                 preferred_element_type=jnp.float32) * scale
            s2 = lax.dot_general(q, k_other[0, :, head, :], dn_qk,
                                 preferred_element_type=jnp.float32) * scale
            m = jnp.maximum(jnp.max(s1, axis=1, keepdims=True),
                            jnp.max(s2, axis=1, keepdims=True))
            p1 = jnp.exp(s1 - m)
            p2 = jnp.exp(s2 - m)
            denom = (jnp.sum(p1, axis=1, keepdims=True)
                     + jnp.sum(p2, axis=1, keepdims=True))
            o = lax.dot_general(p1, v_ref[0, :, head, :], dn_pv,
                                preferred_element_type=jnp.float32)
            o = o + lax.dot_general(p2, v_other[0, :, head, :], dn_pv,
                                    preferred_element_type=jnp.float32)
            out_ref[0, :, head, :] = o / denom

    return pl.pallas_call(
        body,
        out_shape=jax.ShapeDtypeStruct((b, s, h, d), jnp.float32),
        in_specs=[pl.BlockSpec(memory_space=pltpu.VMEM)] * 3,
        out_specs=pl.BlockSpec(memory_space=pltpu.VMEM),
        scratch_shapes=[
            pltpu.VMEM((b, s, h, d), jnp.float32),
            pltpu.VMEM((b, s, h, d), jnp.float32),
            pltpu.SemaphoreType.DMA((2,)),
            pltpu.SemaphoreType.DMA((2,)),
        ],
        compiler_params=pltpu.CompilerParams(collective_id=0),
    )(Q, K, V)


# baseline (device time: 309985 ns/iter reference)
import jax
import jax.numpy as jnp
from jax import lax
from jax.experimental import pallas as pl
from jax.experimental.pallas import tpu as pltpu


def _exchange_kv(K, V):

    def body(k_ref, v_ref, k_other, v_other, send_sems, recv_sems):
        my_x = lax.axis_index("x")
        my_y = lax.axis_index("y")
        peer = (1 - my_x, my_y)

        barrier_sem = pltpu.get_barrier_semaphore()
        pl.semaphore_signal(
            barrier_sem, inc=1, device_id=peer,
            device_id_type=pl.DeviceIdType.MESH,
        )
        pl.semaphore_wait(barrier_sem, 1)

        rdma_k = pltpu.make_async_remote_copy(
            src_ref=k_ref, dst_ref=k_other,
            send_sem=send_sems.at[0], recv_sem=recv_sems.at[0],
            device_id=peer, device_id_type=pl.DeviceIdType.MESH,
        )
        rdma_v = pltpu.make_async_remote_copy(
            src_ref=v_ref, dst_ref=v_other,
            send_sem=send_sems.at[1], recv_sem=recv_sems.at[1],
            device_id=peer, device_id_type=pl.DeviceIdType.MESH,
        )
        rdma_k.start()
        rdma_v.start()
        rdma_k.wait()
        rdma_v.wait()

    return pl.pallas_call(
        body,
        out_shape=[
            jax.ShapeDtypeStruct(K.shape, K.dtype),
            jax.ShapeDtypeStruct(V.shape, V.dtype),
        ],
        in_specs=[
            pl.BlockSpec(memory_space=pl.ANY),
            pl.BlockSpec(memory_space=pl.ANY),
        ],
        out_specs=[
            pl.BlockSpec(memory_space=pl.ANY),
            pl.BlockSpec(memory_space=pl.ANY),
        ],
        scratch_shapes=[
            pltpu.SemaphoreType.DMA((2,)),
            pltpu.SemaphoreType.DMA((2,)),
        ],
        compiler_params=pltpu.CompilerParams(collective_id=0),
    )(K, V)


def kernel(Q, K, V):
    b, s, h, d = Q.shape
    scale = d ** -0.5

    Q2 = Q.reshape(s, h * d)
    K2 = K.reshape(s, h * d)
    V2 = V.reshape(s, h * d)

    Ko, Vo = _exchange_kv(K2, V2)

    dn_qk = (((1,), (1,)), ((), ()))
    dn_pv = (((1,), (0,)), ((), ()))

    def body(q_ref, k_ref, v_ref, ko_ref, vo_ref, out_ref):
        q = q_ref[...]
        s1 = lax.dot_general(q, k_ref[...], dn_qk,
                             preferred_element_type=jnp.float32) * scale
        s2 = lax.dot_general(q, ko_ref[...], dn_qk,
                             preferred_element_type=jnp.float32) * scale
        m = jnp.maximum(jnp.max(s1, axis=1, keepdims=True),
                        jnp.max(s2, axis=1, keepdims=True))
        p1 = jnp.exp(s1 - m)
        p2 = jnp.exp(s2 - m)
        denom = (jnp.sum(p1, axis=1, keepdims=True)
                 + jnp.sum(p2, axis=1, keepdims=True))
        o = lax.dot_general(p1, v_ref[...], dn_pv,
                            preferred_element_type=jnp.float32)
        o = o + lax.dot_general(p2, vo_ref[...], dn_pv,
                                preferred_element_type=jnp.float32)
        out_ref[...] = o / denom

    head_spec = pl.BlockSpec((s, d), lambda i: (0, i))
    out2 = pl.pallas_call(
        body,
        grid=(h,),
        out_shape=jax.ShapeDtypeStruct((s, h * d), jnp.float32),
        in_specs=[head_spec] * 5,
        out_specs=head_spec,
    )(Q2, K2, V2, Ko, Vo)
    return out2.reshape(b, s, h, d)
